# baseline (speedup 1.0000x reference)
import functools

import jax
import jax.numpy as jnp
from jax import lax
from jax.experimental import pallas as pl
from jax.experimental.pallas import tpu as pltpu
from jax.experimental.pallas import tpu_sc as plsc


def _sc_workers():
    try:
        info = plsc.get_sparse_core_info()
        return info.num_cores, info.num_subcores
    except Exception:
        return 2, 16


def kernel(adjacency_fi, embedding_i, emb_f, u, W_w, W_b):
    F, K = adjacency_fi.shape
    N, D = embedding_i.shape
    L = 16
    NC, NS = _sc_workers()
    NW = NC * NS
    B = 2
    IDX = B * K
    CH = D // L
    KG = K // L

    NBUF = 4
    step = NW * B * NBUF
    F_pad = ((F + step - 1) // step) * step
    PW = F_pad // NW
    G = PW // B

    BS = 2048

    def mv_body(e_ref, u_ref, o_ref):
        o_ref[...] = jnp.sum(e_ref[...] * u_ref[...], axis=1, keepdims=True)

    v = pl.pallas_call(
        mv_body,
        grid=(pl.cdiv(N, BS),),
        in_specs=[
            pl.BlockSpec((BS, D), lambda i: (i, 0)),
            pl.BlockSpec((1, D), lambda i: (0, 0)),
        ],
        out_specs=pl.BlockSpec((BS, 1), lambda i: (i, 0)),
        out_shape=jax.ShapeDtypeStruct((N, 1), jnp.float32),
    )(embedding_i, u.reshape(1, D))
    v_flat = v.reshape(-1)

    adj_flat = jnp.pad(
        adjacency_fi.reshape(-1).astype(jnp.int32), (0, (F_pad - F) * K)
    )

    mesh = plsc.VectorSubcoreMesh(core_axis_name="c", subcore_axis_name="s")

    @functools.partial(
        pl.kernel,
        out_type=jax.ShapeDtypeStruct((F_pad * D,), jnp.float32),
        mesh=mesh,
        compiler_params=pltpu.CompilerParams(needs_layout_passes=False),
        scratch_types=(
            [pltpu.VMEM((PW * K,), jnp.int32)]
            + [pltpu.VMEM((IDX, D), jnp.float32)] * NBUF
            + [pltpu.VMEM((IDX,), jnp.float32)] * NBUF
            + [pltpu.VMEM((PW * D,), jnp.float32)]
            + [pltpu.SemaphoreType.DMA] * (2 * NBUF)
        ),
    )
    def sc_agg(adj_hbm, table_hbm, v_hbm, out_hbm, idx_all, *rest):
        rows_b = rest[0:NBUF]
        vals_b = rest[NBUF:2 * NBUF]
        out_all = rest[2 * NBUF]
        sr_b = rest[2 * NBUF + 1:3 * NBUF + 1]
        sv_b = rest[3 * NBUF + 1:4 * NBUF + 1]
        wid = lax.axis_index("s") * NC + lax.axis_index("c")
        wbase = wid * PW
        pltpu.sync_copy(adj_hbm.at[pl.ds(wbase * K, PW * K)], idx_all)

        def start(g, p):
            isl = idx_all.at[pl.ds(g * IDX, IDX)]
            pltpu.async_copy(table_hbm.at[isl], rows_b[p], sr_b[p])
            pltpu.async_copy(v_hbm.at[isl], vals_b[p], sv_b[p])

        def wait(g, p):
            isl = idx_all.at[pl.ds(g * IDX, IDX)]
            pltpu.make_async_copy(table_hbm.at[isl], rows_b[p], sr_b[p]).wait()
            pltpu.make_async_copy(v_hbm.at[isl], vals_b[p], sv_b[p]).wait()

        def compute(g, p):
            rows, vals = rows_b[p], vals_b[p]
            for f in range(B):
                base = f * K
                iv = [idx_all[pl.ds(g * IDX + base + j * L, L)] for j in range(KG)]
                av = [vals[pl.ds(base + j * L, L)] for j in range(KG)]
                mv = [jnp.where(iv[j] == 0, av[j] - 10000.0, av[j])
                      for j in range(KG)]
                m = mv[0]
                for j in range(1, KG):
                    m = jnp.maximum(m, mv[j])
                s = jnp.max(m)
                ev = [jnp.exp(x - s) for x in mv]
                tot = ev[0]
                for j in range(1, KG):
                    tot = tot + ev[j]
                denom = jnp.sum(tot)
                inv = 1.0 / jnp.broadcast_to(denom, (L,))
                accs = [jnp.zeros((L,), jnp.float32) for _ in range(CH)]
                for k in range(K):
                    wk = ev[k // L][k % L]
                    r = base + k
                    for c in range(CH):
                        accs[c] = accs[c] + wk * rows[r, pl.ds(c * L, L)]
                orow = (g * B + f) * D
                for c in range(CH):
                    out_all[pl.ds(orow + c * L, L)] = accs[c] * inv

        for b in range(NBUF):
            start(b, b)

        def body(t, carry):
            g0 = NBUF * t
            for b in range(NBUF):
                g = g0 + b
                wait(g, b)
                compute(g, b)
                start(g + NBUF, b)
            return carry

        lax.fori_loop(0, G // NBUF - 1, body, 0)
        for b in range(NBUF):
            g = G - NBUF + b
            wait(g, b)
            compute(g, b)

        pltpu.sync_copy(out_all, out_hbm.at[pl.ds(wbase * D, PW * D)])

    agg = sc_agg(adj_flat, embedding_i, v_flat).reshape(F_pad, D)[:F]

    BSG = 2000

    def gate_body(e_ref, a_ref, w_ref, b_ref, o_ref):
        e = e_ref[...]
        a = a_ref[...]
        w = w_ref[...]
        z = lax.dot_general(e, w[:, :D], (((1,), (1,)), ((), ())),
                            preferred_element_type=jnp.float32)
        z = z + lax.dot_general(a, w[:, D:], (((1,), (1,)), ((), ())),
                                preferred_element_type=jnp.float32)
        gate = jax.nn.sigmoid(z + b_ref[...])
        o_ref[...] = gate * e + (1.0 - gate) * a

    out = pl.pallas_call(
        gate_body,
        grid=(pl.cdiv(F, BSG),),
        in_specs=[
            pl.BlockSpec((BSG, D), lambda i: (i, 0)),
            pl.BlockSpec((BSG, D), lambda i: (i, 0)),
            pl.BlockSpec((D, 2 * D), lambda i: (0, 0)),
            pl.BlockSpec((1, D), lambda i: (0, 0)),
        ],
        out_specs=pl.BlockSpec((BSG, D), lambda i: (i, 0)),
        out_shape=jax.ShapeDtypeStruct((F, D), jnp.float32),
    )(emb_f, agg, W_w, W_b.reshape(1, D))
    return out


# speedup vs baseline: 1.6670x; 1.6670x over previous
import functools

import jax
import jax.numpy as jnp
from jax import lax
from jax.experimental import pallas as pl
from jax.experimental.pallas import tpu as pltpu
from jax.experimental.pallas import tpu_sc as plsc


def _sc_workers():
    try:
        info = plsc.get_sparse_core_info()
        return info.num_cores, info.num_subcores
    except Exception:
        return 2, 16


def kernel(adjacency_fi, embedding_i, emb_f, u, W_w, W_b):
    F, K = adjacency_fi.shape
    N, D = embedding_i.shape
    L = 16
    NC, NS = _sc_workers()
    NW = NC * NS
    B = 2
    IDX = B * K
    CH = D // L
    KG = K // L

    NBUF = 2
    G_TOT = -(-F // (NS * B))
    G0 = (int(G_TOT * 0.70) // NBUF) * NBUF
    G1 = -(-(G_TOT - G0) // NBUF) * NBUF
    PW0, PW1 = B * G0, B * G1
    F_pad = NS * (PW0 + PW1)

    BS = 2048

    def mv_body(e_ref, u_ref, o_ref):
        o_ref[...] = jnp.sum(e_ref[...] * u_ref[...], axis=1, keepdims=True)

    v = pl.pallas_call(
        mv_body,
        grid=(pl.cdiv(N, BS),),
        in_specs=[
            pl.BlockSpec((BS, D), lambda i: (i, 0)),
            pl.BlockSpec((1, D), lambda i: (0, 0)),
        ],
        out_specs=pl.BlockSpec((BS, 1), lambda i: (i, 0)),
        out_shape=jax.ShapeDtypeStruct((N, 1), jnp.float32),
    )(embedding_i, u.reshape(1, D))
    v_flat = v.reshape(-1)

    adj_flat = jnp.pad(
        adjacency_fi.reshape(-1).astype(jnp.int32), (0, (F_pad - F) * K)
    )

    mesh = plsc.VectorSubcoreMesh(core_axis_name="c", subcore_axis_name="s")

    @functools.partial(
        pl.kernel,
        out_type=jax.ShapeDtypeStruct((F_pad * D,), jnp.float32),
        mesh=mesh,
        compiler_params=pltpu.CompilerParams(needs_layout_passes=False),
        scratch_types=(
            [pltpu.VMEM((PW0 * K,), jnp.int32)]
            + [pltpu.VMEM((IDX, D), jnp.float32)] * NBUF
            + [pltpu.VMEM((IDX,), jnp.float32)] * NBUF
            + [pltpu.VMEM((PW0 * D,), jnp.float32)]
            + [pltpu.SemaphoreType.DMA] * (2 * NBUF)
        ),
    )
    def sc_agg(adj_hbm, table_hbm, v_hbm, out_hbm, idx_all, *rest):
        rows_b = rest[0:NBUF]
        vals_b = rest[NBUF:2 * NBUF]
        out_all = rest[2 * NBUF]
        sr_b = rest[2 * NBUF + 1:3 * NBUF + 1]
        sv_b = rest[3 * NBUF + 1:4 * NBUF + 1]
        cid = lax.axis_index("c")
        sid = lax.axis_index("s")

        def start(g, p):
            isl = idx_all.at[pl.ds(g * IDX, IDX)]
            pltpu.async_copy(table_hbm.at[isl], rows_b[p], sr_b[p])
            pltpu.async_copy(v_hbm.at[isl], vals_b[p], sv_b[p])

        def wait(g, p):
            isl = idx_all.at[pl.ds(g * IDX, IDX)]
            pltpu.make_async_copy(table_hbm.at[isl], rows_b[p], sr_b[p]).wait()
            pltpu.make_async_copy(v_hbm.at[isl], vals_b[p], sv_b[p]).wait()

        def compute(g, p):
            rows, vals = rows_b[p], vals_b[p]
            for f in range(B):
                base = f * K
                iv = [idx_all[pl.ds(g * IDX + base + j * L, L)] for j in range(KG)]
                av = [vals[pl.ds(base + j * L, L)] for j in range(KG)]
                mv = [jnp.where(iv[j] == 0, av[j] - 10000.0, av[j])
                      for j in range(KG)]
                m = mv[0]
                for j in range(1, KG):
                    m = jnp.maximum(m, mv[j])
                s = jnp.max(m)
                ev = [jnp.exp(x - s) for x in mv]
                tot = ev[0]
                for j in range(1, KG):
                    tot = tot + ev[j]
                denom = jnp.sum(tot)
                inv = 1.0 / jnp.broadcast_to(denom, (L,))
                accs = [jnp.zeros((L,), jnp.float32) for _ in range(CH)]
                for k in range(K):
                    wk = ev[k // L][k % L]
                    r = base + k
                    for c in range(CH):
                        accs[c] = accs[c] + wk * rows[r, pl.ds(c * L, L)]
                orow = (g * B + f) * D
                for c in range(CH):
                    out_all[pl.ds(orow + c * L, L)] = accs[c] * inv

        def pipeline(G, PW, wbase):
            pltpu.sync_copy(adj_hbm.at[pl.ds(wbase * K, PW * K)],
                            idx_all.at[pl.ds(0, PW * K)])
            for b in range(NBUF):
                start(b, b)

            def body(t, carry):
                g0 = NBUF * t
                for b in range(NBUF):
                    g = g0 + b
                    wait(g, b)
                    compute(g, b)
                    start(g + NBUF, b)
                return carry

            lax.fori_loop(0, G // NBUF - 1, body, 0)
            for b in range(NBUF):
                g = G - NBUF + b
                wait(g, b)
                compute(g, b)
            pltpu.sync_copy(out_all.at[pl.ds(0, PW * D)],
                            out_hbm.at[pl.ds(wbase * D, PW * D)])

        @pl.when(cid == 0)
        def _():
            pipeline(G0, PW0, sid * PW0)

        @pl.when(cid == 1)
        def _():
            pipeline(G1, PW1, NS * PW0 + sid * PW1)

    agg = sc_agg(adj_flat, embedding_i, v_flat).reshape(F_pad, D)[:F]

    BSG = 2000

    def gate_body(e_ref, a_ref, w_ref, b_ref, o_ref):
        e = e_ref[...]
        a = a_ref[...]
        w = w_ref[...]
        z = lax.dot_general(e, w[:, :D], (((1,), (1,)), ((), ())),
                            preferred_element_type=jnp.float32)
        z = z + lax.dot_general(a, w[:, D:], (((1,), (1,)), ((), ())),
                                preferred_element_type=jnp.float32)
        gate = jax.nn.sigmoid(z + b_ref[...])
        o_ref[...] = gate * e + (1.0 - gate) * a

    out = pl.pallas_call(
        gate_body,
        grid=(pl.cdiv(F, BSG),),
        in_specs=[
            pl.BlockSpec((BSG, D), lambda i: (i, 0)),
            pl.BlockSpec((BSG, D), lambda i: (i, 0)),
            pl.BlockSpec((D, 2 * D), lambda i: (0, 0)),
            pl.BlockSpec((1, D), lambda i: (0, 0)),
        ],
        out_specs=pl.BlockSpec((BSG, D), lambda i: (i, 0)),
        out_shape=jax.ShapeDtypeStruct((F, D), jnp.float32),
    )(emb_f, agg, W_w, W_b.reshape(1, D))
    return out


# speedup vs baseline: 1.8336x; 1.0999x over previous
import functools

import jax
import jax.numpy as jnp
from jax import lax
from jax.experimental import pallas as pl
from jax.experimental.pallas import tpu as pltpu
from jax.experimental.pallas import tpu_sc as plsc


def _sc_workers():
    try:
        info = plsc.get_sparse_core_info()
        return info.num_cores, info.num_subcores
    except Exception:
        return 2, 16


def kernel(adjacency_fi, embedding_i, emb_f, u, W_w, W_b):
    F, K = adjacency_fi.shape
    N, D = embedding_i.shape
    L = 16
    NC, NS = _sc_workers()
    NW = NC * NS
    B = 2
    IDX = B * K
    CH = D // L
    KG = K // L

    NBUF = 2
    G_TOT = -(-F // (NS * B))
    G0 = (int(G_TOT * 0.64) // NBUF) * NBUF
    G1 = -(-(G_TOT - G0) // NBUF) * NBUF
    PW0, PW1 = B * G0, B * G1
    F_pad = NS * (PW0 + PW1)

    BS = 2048

    def mv_body(e_ref, u_ref, o_ref):
        o_ref[...] = jnp.sum(e_ref[...] * u_ref[...], axis=1)

    v_flat = pl.pallas_call(
        mv_body,
        grid=(pl.cdiv(N, BS),),
        in_specs=[
            pl.BlockSpec((BS, D), lambda i: (i, 0)),
            pl.BlockSpec((1, D), lambda i: (0, 0)),
        ],
        out_specs=pl.BlockSpec((BS,), lambda i: (i,)),
        out_shape=jax.ShapeDtypeStruct((N,), jnp.float32),
    )(embedding_i, u.reshape(1, D))

    adj_flat = jnp.pad(
        adjacency_fi.reshape(-1).astype(jnp.int32), (0, (F_pad - F) * K)
    )

    mesh = plsc.VectorSubcoreMesh(core_axis_name="c", subcore_axis_name="s",
                                  num_cores=NC, num_subcores=NS)

    @functools.partial(
        pl.kernel,
        out_type=jax.ShapeDtypeStruct((F_pad * D,), jnp.float32),
        mesh=mesh,
        compiler_params=pltpu.CompilerParams(needs_layout_passes=False),
        scratch_types=(
            [pltpu.VMEM((PW0 * K,), jnp.int32)]
            + [pltpu.VMEM((IDX, D), jnp.float32)] * NBUF
            + [pltpu.VMEM((IDX,), jnp.float32)] * NBUF
            + [pltpu.VMEM((PW0 * D,), jnp.float32)]
            + [pltpu.SemaphoreType.DMA] * (2 * NBUF)
        ),
    )
    def sc_agg(adj_hbm, table_hbm, v_hbm, out_hbm, idx_all, *rest):
        rows_b = rest[0:NBUF]
        vals_b = rest[NBUF:2 * NBUF]
        out_all = rest[2 * NBUF]
        sr_b = rest[2 * NBUF + 1:3 * NBUF + 1]
        sv_b = rest[3 * NBUF + 1:4 * NBUF + 1]
        cid = lax.axis_index("c")
        sid = lax.axis_index("s")

        def start(g, p):
            isl = idx_all.at[pl.ds(g * IDX, IDX)]
            pltpu.async_copy(table_hbm.at[isl], rows_b[p], sr_b[p])
            pltpu.async_copy(v_hbm.at[isl], vals_b[p], sv_b[p])

        def wait(g, p):
            isl = idx_all.at[pl.ds(g * IDX, IDX)]
            pltpu.make_async_copy(table_hbm.at[isl], rows_b[p], sr_b[p]).wait()
            pltpu.make_async_copy(v_hbm.at[isl], vals_b[p], sv_b[p]).wait()

        def compute(g, p):
            rows, vals = rows_b[p], vals_b[p]
            for f in range(B):
                base = f * K
                iv = [idx_all[pl.ds(g * IDX + base + j * L, L)] for j in range(KG)]
                av = [vals[pl.ds(base + j * L, L)] for j in range(KG)]
                mv = [jnp.where(iv[j] == 0, av[j] - 10000.0, av[j])
                      for j in range(KG)]
                m = mv[0]
                for j in range(1, KG):
                    m = jnp.maximum(m, mv[j])
                s = jnp.max(m)
                ev = [jnp.exp(x - s) for x in mv]
                tot = ev[0]
                for j in range(1, KG):
                    tot = tot + ev[j]
                denom = jnp.sum(tot)
                inv = 1.0 / jnp.broadcast_to(denom, (L,))
                accs = [jnp.zeros((L,), jnp.float32) for _ in range(CH)]
                for k in range(K):
                    wk = ev[k // L][k % L]
                    r = base + k
                    for c in range(CH):
                        accs[c] = accs[c] + wk * rows[r, pl.ds(c * L, L)]
                orow = (g * B + f) * D
                for c in range(CH):
                    out_all[pl.ds(orow + c * L, L)] = accs[c] * inv

        def pipeline(G, PW, wbase):
            pltpu.sync_copy(adj_hbm.at[pl.ds(wbase * K, PW * K)],
                            idx_all.at[pl.ds(0, PW * K)])
            for b in range(NBUF):
                start(b, b)

            def body(t, carry):
                g0 = NBUF * t
                for b in range(NBUF):
                    g = g0 + b
                    wait(g, b)
                    compute(g, b)
                    start(g + NBUF, b)
                return carry

            lax.fori_loop(0, G // NBUF - 1, body, 0)
            for b in range(NBUF):
                g = G - NBUF + b
                wait(g, b)
                compute(g, b)
            pltpu.sync_copy(out_all.at[pl.ds(0, PW * D)],
                            out_hbm.at[pl.ds(wbase * D, PW * D)])

        @pl.when(cid == 0)
        def _():
            pipeline(G0, PW0, sid * PW0)

        @pl.when(cid == 1)
        def _():
            pipeline(G1, PW1, NS * PW0 + sid * PW1)

    agg = sc_agg(adj_flat, embedding_i, v_flat).reshape(F_pad, D)[:F]

    BSG = 2000

    def gate_body(e_ref, a_ref, w_ref, b_ref, o_ref):
        e = e_ref[...]
        a = a_ref[...]
        w = w_ref[...]
        z = lax.dot_general(e, w[:, :D], (((1,), (1,)), ((), ())),
                            preferred_element_type=jnp.float32)
        z = z + lax.dot_general(a, w[:, D:], (((1,), (1,)), ((), ())),
                                preferred_element_type=jnp.float32)
        gate = jax.nn.sigmoid(z + b_ref[...])
        o_ref[...] = gate * e + (1.0 - gate) * a

    out = pl.pallas_call(
        gate_body,
        grid=(pl.cdiv(F, BSG),),
        in_specs=[
            pl.BlockSpec((BSG, D), lambda i: (i, 0)),
            pl.BlockSpec((BSG, D), lambda i: (i, 0)),
            pl.BlockSpec((D, 2 * D), lambda i: (0, 0)),
            pl.BlockSpec((1, D), lambda i: (0, 0)),
        ],
        out_specs=pl.BlockSpec((BSG, D), lambda i: (i, 0)),
        out_shape=jax.ShapeDtypeStruct((F, D), jnp.float32),
    )(emb_f, agg, W_w, W_b.reshape(1, D))
    return out


# speedup vs baseline: 1.9424x; 1.0593x over previous
import functools

import jax
import jax.numpy as jnp
from jax import lax
from jax.experimental import pallas as pl
from jax.experimental.pallas import tpu as pltpu
from jax.experimental.pallas import tpu_sc as plsc


def _sc_workers():
    try:
        info = plsc.get_sparse_core_info()
        return info.num_cores, info.num_subcores
    except Exception:
        return 2, 16


def kernel(adjacency_fi, embedding_i, emb_f, u, W_w, W_b):
    F, K = adjacency_fi.shape
    N, D = embedding_i.shape
    L = 16
    NC, NS = _sc_workers()
    NW = NC * NS
    B = 2
    IDX = B * K
    CH = D // L
    KG = K // L

    NBUF = 2
    G_TOT = -(-F // (NS * B))
    G0 = (int(G_TOT * 0.61) // NBUF) * NBUF
    G1 = -(-(G_TOT - G0) // NBUF) * NBUF
    PW0, PW1 = B * G0, B * G1
    F_pad = NS * (PW0 + PW1)

    BS = 4096

    def mv_body(e_ref, u_ref, o_ref):
        d = lax.dot_general(e_ref[...], u_ref[...], (((1,), (1,)), ((), ())),
                            preferred_element_type=jnp.float32)
        o_ref[...] = d[:, 0]

    v_flat = pl.pallas_call(
        mv_body,
        grid=(pl.cdiv(N, BS),),
        in_specs=[
            pl.BlockSpec((BS, D), lambda i: (i, 0)),
            pl.BlockSpec((1, D), lambda i: (0, 0)),
        ],
        out_specs=pl.BlockSpec((BS,), lambda i: (i,)),
        out_shape=jax.ShapeDtypeStruct((N,), jnp.float32),
    )(embedding_i, u.reshape(1, D))

    adj_flat = jnp.pad(
        adjacency_fi.reshape(-1).astype(jnp.int32), (0, (F_pad - F) * K)
    )

    mesh = plsc.VectorSubcoreMesh(core_axis_name="c", subcore_axis_name="s",
                                  num_cores=NC, num_subcores=NS)

    @functools.partial(
        pl.kernel,
        out_type=jax.ShapeDtypeStruct((F_pad * D,), jnp.float32),
        mesh=mesh,
        compiler_params=pltpu.CompilerParams(needs_layout_passes=False),
        scratch_types=(
            [pltpu.VMEM((PW0 * K,), jnp.int32)]
            + [pltpu.VMEM((IDX, D), jnp.float32)] * NBUF
            + [pltpu.VMEM((IDX,), jnp.float32)] * NBUF
            + [pltpu.VMEM((PW0 * D,), jnp.float32)]
            + [pltpu.SemaphoreType.DMA] * (2 * NBUF)
        ),
    )
    def sc_agg(adj_hbm, table_hbm, v_hbm, out_hbm, idx_all, *rest):
        rows_b = rest[0:NBUF]
        vals_b = rest[NBUF:2 * NBUF]
        out_all = rest[2 * NBUF]
        sr_b = rest[2 * NBUF + 1:3 * NBUF + 1]
        sv_b = rest[3 * NBUF + 1:4 * NBUF + 1]
        cid = lax.axis_index("c")
        sid = lax.axis_index("s")

        def start(g, p):
            isl = idx_all.at[pl.ds(g * IDX, IDX)]
            pltpu.async_copy(table_hbm.at[isl], rows_b[p], sr_b[p])
            pltpu.async_copy(v_hbm.at[isl], vals_b[p], sv_b[p])

        def wait(g, p):
            isl = idx_all.at[pl.ds(g * IDX, IDX)]
            pltpu.make_async_copy(table_hbm.at[isl], rows_b[p], sr_b[p]).wait()
            pltpu.make_async_copy(v_hbm.at[isl], vals_b[p], sv_b[p]).wait()

        def compute(g, p):
            rows, vals = rows_b[p], vals_b[p]
            for f in range(B):
                base = f * K
                iv = [idx_all[pl.ds(g * IDX + base + j * L, L)] for j in range(KG)]
                av = [vals[pl.ds(base + j * L, L)] for j in range(KG)]
                mv = [jnp.where(iv[j] == 0, av[j] - 10000.0, av[j])
                      for j in range(KG)]
                m = mv[0]
                for j in range(1, KG):
                    m = jnp.maximum(m, mv[j])
                s = jnp.max(m)
                ev = [jnp.exp(x - s) for x in mv]
                tot = ev[0]
                for j in range(1, KG):
                    tot = tot + ev[j]
                denom = jnp.sum(tot)
                inv = 1.0 / jnp.broadcast_to(denom, (L,))
                accs = [jnp.zeros((L,), jnp.float32) for _ in range(CH)]
                for k in range(K):
                    wk = ev[k // L][k % L]
                    r = base + k
                    for c in range(CH):
                        accs[c] = accs[c] + wk * rows[r, pl.ds(c * L, L)]
                orow = (g * B + f) * D
                for c in range(CH):
                    out_all[pl.ds(orow + c * L, L)] = accs[c] * inv

        def pipeline(G, PW, wbase):
            pltpu.sync_copy(adj_hbm.at[pl.ds(wbase * K, PW * K)],
                            idx_all.at[pl.ds(0, PW * K)])
            for b in range(NBUF):
                start(b, b)

            def body(t, carry):
                g0 = NBUF * t
                for b in range(NBUF):
                    g = g0 + b
                    wait(g, b)
                    compute(g, b)
                    start(g + NBUF, b)
                return carry

            lax.fori_loop(0, G // NBUF - 1, body, 0)
            for b in range(NBUF):
                g = G - NBUF + b
                wait(g, b)
                compute(g, b)
            pltpu.sync_copy(out_all.at[pl.ds(0, PW * D)],
                            out_hbm.at[pl.ds(wbase * D, PW * D)])

        @pl.when(cid == 0)
        def _():
            pipeline(G0, PW0, sid * PW0)

        @pl.when(cid == 1)
        def _():
            pipeline(G1, PW1, NS * PW0 + sid * PW1)

    agg = sc_agg(adj_flat, embedding_i, v_flat).reshape(F_pad, D)[:F]

    BSG = 2000

    def gate_body(e_ref, a_ref, w_ref, b_ref, o_ref):
        e = e_ref[...]
        a = a_ref[...]
        w = w_ref[...]
        z = lax.dot_general(e, w[:, :D], (((1,), (1,)), ((), ())),
                            preferred_element_type=jnp.float32)
        z = z + lax.dot_general(a, w[:, D:], (((1,), (1,)), ((), ())),
                                preferred_element_type=jnp.float32)
        gate = jax.nn.sigmoid(z + b_ref[...])
        o_ref[...] = gate * e + (1.0 - gate) * a

    out = pl.pallas_call(
        gate_body,
        grid=(pl.cdiv(F, BSG),),
        in_specs=[
            pl.BlockSpec((BSG, D), lambda i: (i, 0)),
            pl.BlockSpec((BSG, D), lambda i: (i, 0)),
            pl.BlockSpec((D, 2 * D), lambda i: (0, 0)),
            pl.BlockSpec((1, D), lambda i: (0, 0)),
        ],
        out_specs=pl.BlockSpec((BSG, D), lambda i: (i, 0)),
        out_shape=jax.ShapeDtypeStruct((F, D), jnp.float32),
    )(emb_f, agg, W_w, W_b.reshape(1, D))
    return out


# speedup vs baseline: 1.9771x; 1.0179x over previous
import functools

import jax
import jax.numpy as jnp
from jax import lax
from jax.experimental import pallas as pl
from jax.experimental.pallas import tpu as pltpu
from jax.experimental.pallas import tpu_sc as plsc


def _sc_workers():
    try:
        info = plsc.get_sparse_core_info()
        return info.num_cores, info.num_subcores
    except Exception:
        return 2, 16


def kernel(adjacency_fi, embedding_i, emb_f, u, W_w, W_b):
    F, K = adjacency_fi.shape
    N, D = embedding_i.shape
    L = 16
    NC, NS = _sc_workers()
    NW = NC * NS
    B = 2
    IDX = B * K
    CH = D // L
    KG = K // L

    NBUF = 2
    G_TOT = -(-F // (NS * B))
    G0 = (int(G_TOT * 0.61) // NBUF) * NBUF
    G1 = -(-(G_TOT - G0) // NBUF) * NBUF
    PW0, PW1 = B * G0, B * G1
    F_pad = NS * (PW0 + PW1)

    BS = 8192

    def mv_body(e_ref, u_ref, o_ref):
        d = lax.dot_general(e_ref[...], u_ref[...], (((1,), (1,)), ((), ())),
                            preferred_element_type=jnp.float32)
        o_ref[...] = d[:, 0]

    v_flat = pl.pallas_call(
        mv_body,
        grid=(pl.cdiv(N, BS),),
        in_specs=[
            pl.BlockSpec((BS, D), lambda i: (i, 0)),
            pl.BlockSpec((1, D), lambda i: (0, 0)),
        ],
        out_specs=pl.BlockSpec((BS,), lambda i: (i,)),
        out_shape=jax.ShapeDtypeStruct((N,), jnp.float32),
    )(embedding_i, u.reshape(1, D))

    adj_flat = jnp.pad(
        adjacency_fi.reshape(-1).astype(jnp.int32), (0, (F_pad - F) * K)
    )

    mesh = plsc.VectorSubcoreMesh(core_axis_name="c", subcore_axis_name="s",
                                  num_cores=NC, num_subcores=NS)

    @functools.partial(
        pl.kernel,
        out_type=jax.ShapeDtypeStruct((F_pad * D,), jnp.float32),
        mesh=mesh,
        compiler_params=pltpu.CompilerParams(needs_layout_passes=False),
        scratch_types=(
            [pltpu.VMEM((PW0 * K,), jnp.int32)]
            + [pltpu.VMEM((IDX, D), jnp.float32)] * NBUF
            + [pltpu.VMEM((IDX,), jnp.float32)] * NBUF
            + [pltpu.VMEM((PW0 * D,), jnp.float32)]
            + [pltpu.SemaphoreType.DMA] * (2 * NBUF)
        ),
    )
    def sc_agg(adj_hbm, table_hbm, v_hbm, out_hbm, idx_all, *rest):
        rows_b = rest[0:NBUF]
        vals_b = rest[NBUF:2 * NBUF]
        out_all = rest[2 * NBUF]
        sr_b = rest[2 * NBUF + 1:3 * NBUF + 1]
        sv_b = rest[3 * NBUF + 1:4 * NBUF + 1]
        cid = lax.axis_index("c")
        sid = lax.axis_index("s")

        def start(g, p):
            isl = idx_all.at[pl.ds(g * IDX, IDX)]
            pltpu.async_copy(table_hbm.at[isl], rows_b[p], sr_b[p])
            pltpu.async_copy(v_hbm.at[isl], vals_b[p], sv_b[p])

        def wait(g, p):
            isl = idx_all.at[pl.ds(g * IDX, IDX)]
            pltpu.make_async_copy(table_hbm.at[isl], rows_b[p], sr_b[p]).wait()
            pltpu.make_async_copy(v_hbm.at[isl], vals_b[p], sv_b[p]).wait()

        def compute(g, p):
            rows, vals = rows_b[p], vals_b[p]
            for f in range(B):
                base = f * K
                iv = [idx_all[pl.ds(g * IDX + base + j * L, L)] for j in range(KG)]
                av = [vals[pl.ds(base + j * L, L)] for j in range(KG)]
                mv = [jnp.where(iv[j] == 0, av[j] - 10000.0, av[j])
                      for j in range(KG)]
                m = mv[0]
                for j in range(1, KG):
                    m = jnp.maximum(m, mv[j])
                s = jnp.max(m)
                ev = [jnp.exp(x - s) for x in mv]
                tot = ev[0]
                for j in range(1, KG):
                    tot = tot + ev[j]
                denom = jnp.sum(tot)
                inv = 1.0 / jnp.broadcast_to(denom, (L,))
                accs = [jnp.zeros((L,), jnp.float32) for _ in range(CH)]
                for k in range(K):
                    wk = ev[k // L][k % L]
                    r = base + k
                    for c in range(CH):
                        accs[c] = accs[c] + wk * rows[r, pl.ds(c * L, L)]
                orow = (g * B + f) * D
                for c in range(CH):
                    out_all[pl.ds(orow + c * L, L)] = accs[c] * inv

        def pipeline(G, PW, wbase):
            pltpu.sync_copy(adj_hbm.at[pl.ds(wbase * K, PW * K)],
                            idx_all.at[pl.ds(0, PW * K)])
            for b in range(NBUF):
                start(b, b)

            def body(t, carry):
                g0 = NBUF * t
                for b in range(NBUF):
                    g = g0 + b
                    wait(g, b)
                    compute(g, b)
                    start(g + NBUF, b)
                return carry

            lax.fori_loop(0, G // NBUF - 1, body, 0)
            for b in range(NBUF):
                g = G - NBUF + b
                wait(g, b)
                compute(g, b)
            pltpu.sync_copy(out_all.at[pl.ds(0, PW * D)],
                            out_hbm.at[pl.ds(wbase * D, PW * D)])

        @pl.when(cid == 0)
        def _():
            pipeline(G0, PW0, sid * PW0)

        @pl.when(cid == 1)
        def _():
            pipeline(G1, PW1, NS * PW0 + sid * PW1)

    agg = sc_agg(adj_flat, embedding_i, v_flat).reshape(F_pad, D)

    BSG = 2000

    def gate_body(e_ref, a_ref, w_ref, b_ref, o_ref):
        e = e_ref[...]
        a = a_ref[...]
        w = w_ref[...]
        z = lax.dot_general(e, w[:, :D], (((1,), (1,)), ((), ())),
                            preferred_element_type=jnp.float32)
        z = z + lax.dot_general(a, w[:, D:], (((1,), (1,)), ((), ())),
                                preferred_element_type=jnp.float32)
        gate = jax.nn.sigmoid(z + b_ref[...])
        o_ref[...] = gate * e + (1.0 - gate) * a

    out = pl.pallas_call(
        gate_body,
        grid=(pl.cdiv(F, BSG),),
        in_specs=[
            pl.BlockSpec((BSG, D), lambda i: (i, 0)),
            pl.BlockSpec((BSG, D), lambda i: (i, 0)),
            pl.BlockSpec((D, 2 * D), lambda i: (0, 0)),
            pl.BlockSpec((1, D), lambda i: (0, 0)),
        ],
        out_specs=pl.BlockSpec((BSG, D), lambda i: (i, 0)),
        out_shape=jax.ShapeDtypeStruct((F, D), jnp.float32),
    )(emb_f, agg, W_w, W_b.reshape(1, D))
    return out
